# Initial kernel scaffold; baseline (speedup 1.0000x reference)
import jax
import jax.numpy as jnp
from jax.experimental import pallas as pl


def kernel(x, edge_index, W1, b1, W2, b2):
    raise NotImplementedError("write your pallas kernel here")


# speedup vs baseline: 23.5561x; 23.5561x over previous
import dataclasses
import functools

import jax
import jax.numpy as jnp
from jax import lax
from jax.experimental import pallas as pl
from jax.experimental.pallas import tpu as pltpu
from jax.experimental.pallas import tpu_sc as plsc

N = 10000
D = 128
NC = 2
NS = 16
L = 16
NW = NC * NS
K = 128
CH = 80
EPW = CH * K
EP = NW * EPW
NPAD = 10240
RPW = NPAD // NS
ZR = 64
BM = 2048

_mesh = plsc.VectorSubcoreMesh(
    core_axis_name="c", subcore_axis_name="s", num_cores=NC, num_subcores=NS
)

_sc_params = pltpu.CompilerParams()
if "needs_layout_passes" in pltpu.CompilerParams.__dataclass_fields__:
    _sc_params = dataclasses.replace(_sc_params, needs_layout_passes=False)


def _deg_body(dst_hbm, deg_hbm, d_v, hist):
    c = lax.axis_index("c")
    s = lax.axis_index("s")
    w = c * NS + s
    zero16 = jnp.zeros((L,), jnp.float32)
    one16 = jnp.ones((L,), jnp.float32)

    @pl.loop(0, NPAD, step=L)
    def _(i):
        hist[pl.ds(i, L)] = zero16

    pltpu.sync_copy(dst_hbm.at[w], d_v)

    @pl.loop(0, CH)
    def _(j):
        @pl.loop(0, K, step=L)
        def _(k):
            idx = d_v[j, pl.ds(k, L)]
            plsc.addupdate_scatter(hist, [idx], one16)

    pltpu.sync_copy(hist, deg_hbm.at[w])


_deg_call = pl.kernel(
    _deg_body,
    out_type=jax.ShapeDtypeStruct((NW, NPAD), jnp.float32),
    mesh=_mesh,
    scratch_types=[
        pltpu.VMEM((CH, K), jnp.int32),
        pltpu.VMEM((NPAD,), jnp.float32),
    ],
    compiler_params=_sc_params,
)


def _edge_body(g_hbm, src_hbm, dst_hbm, part_hbm, s_v, d_v, buf, zbuf, accum,
               gsem):
    c = lax.axis_index("c")
    s = lax.axis_index("s")
    w = c * NS + s
    zero16 = jnp.zeros((L,), jnp.float32)

    @pl.loop(0, ZR)
    def _(r):
        @pl.loop(0, D, step=L)
        def _(k):
            zbuf[r, pl.ds(k, L)] = zero16

    @pl.loop(0, RPW, step=ZR)
    def _(r0):
        pltpu.sync_copy(zbuf, accum.at[pl.ds(s * RPW + r0, ZR)])

    plsc.subcore_barrier()

    pltpu.sync_copy(src_hbm.at[w], s_v)
    pltpu.sync_copy(dst_hbm.at[w], d_v)

    @pl.loop(0, CH)
    def _(j):
        pltpu.async_copy(g_hbm.at[s_v.at[j]], buf, gsem).wait()
        pltpu.sync_copy(buf, accum.at[d_v.at[j]], add=True)

    plsc.subcore_barrier()

    pltpu.sync_copy(
        accum.at[pl.ds(s * RPW, RPW)],
        part_hbm.at[c, pl.ds(s * RPW, RPW)],
    )


_edge_call = pl.kernel(
    _edge_body,
    out_type=jax.ShapeDtypeStruct((NC, NPAD, D), jnp.float32),
    mesh=_mesh,
    scratch_types=[
        pltpu.VMEM((CH, K), jnp.int32),
        pltpu.VMEM((CH, K), jnp.int32),
        pltpu.VMEM((K, D), jnp.float32),
        pltpu.VMEM((ZR, D), jnp.float32),
        pltpu.VMEM_SHARED((NPAD, D), jnp.float32),
        pltpu.SemaphoreType.DMA,
    ],
    compiler_params=_sc_params,
)


def _dis_of(degs_blk):
    deg = jnp.sum(degs_blk, axis=0) + 1.0
    return lax.rsqrt(deg)


def _mm1_body(x_ref, w_ref, degs_ref, g_ref):
    dis = _dis_of(degs_ref[...])
    h = jnp.dot(x_ref[...], w_ref[...], preferred_element_type=jnp.float32)
    g_ref[...] = h * dis[:, None]


_mm1_call = pl.pallas_call(
    _mm1_body,
    grid=(NPAD // BM,),
    in_specs=[
        pl.BlockSpec((BM, D), lambda i: (i, 0)),
        pl.BlockSpec((D, D), lambda i: (0, 0)),
        pl.BlockSpec((NW, BM), lambda i: (0, i)),
    ],
    out_specs=pl.BlockSpec((BM, D), lambda i: (i, 0)),
    out_shape=jax.ShapeDtypeStruct((NPAD, D), jnp.float32),
)


def _mm2_body(p_ref, g1_ref, degs_ref, b1_ref, w2_ref, g2_ref):
    dis = _dis_of(degs_ref[...])
    m = p_ref[0] + p_ref[1] + g1_ref[...]
    a = jnp.maximum(m * dis[:, None] + b1_ref[...], 0.0)
    h2 = jnp.dot(a, w2_ref[...], preferred_element_type=jnp.float32)
    g2_ref[...] = h2 * dis[:, None]


_mm2_call = pl.pallas_call(
    _mm2_body,
    grid=(NPAD // BM,),
    in_specs=[
        pl.BlockSpec((NC, BM, D), lambda i: (0, i, 0)),
        pl.BlockSpec((BM, D), lambda i: (i, 0)),
        pl.BlockSpec((NW, BM), lambda i: (0, i)),
        pl.BlockSpec((1, D), lambda i: (0, 0)),
        pl.BlockSpec((D, D), lambda i: (0, 0)),
    ],
    out_specs=pl.BlockSpec((BM, D), lambda i: (i, 0)),
    out_shape=jax.ShapeDtypeStruct((NPAD, D), jnp.float32),
)


def _out_body(p_ref, g2_ref, degs_ref, b2_ref, o_ref):
    dis = _dis_of(degs_ref[...])
    m = p_ref[0] + p_ref[1] + g2_ref[...]
    o_ref[...] = m * dis[:, None] + b2_ref[...]


_out_call = pl.pallas_call(
    _out_body,
    grid=(NPAD // BM,),
    in_specs=[
        pl.BlockSpec((NC, BM, D), lambda i: (0, i, 0)),
        pl.BlockSpec((BM, D), lambda i: (i, 0)),
        pl.BlockSpec((NW, BM), lambda i: (0, i)),
        pl.BlockSpec((1, D), lambda i: (0, 0)),
    ],
    out_specs=pl.BlockSpec((BM, D), lambda i: (i, 0)),
    out_shape=jax.ShapeDtypeStruct((NPAD, D), jnp.float32),
)


def kernel(x, edge_index, W1, b1, W2, b2):
    e = edge_index.shape[1]
    pad = EP - e
    ar = jnp.arange(pad, dtype=jnp.int32)
    src_p = jnp.concatenate([edge_index[0], ar % N]).reshape(NW, CH, K)
    dst_p = jnp.concatenate([edge_index[1], N + ar % (NPAD - N)]).reshape(
        NW, CH, K)
    x_p = jnp.pad(x, ((0, NPAD - N), (0, 0)))

    degs = _deg_call(dst_p)
    g1 = _mm1_call(x_p, W1, degs)
    p1 = _edge_call(g1, src_p, dst_p)
    g2 = _mm2_call(p1, g1, degs, b1.reshape(1, D), W2)
    p2 = _edge_call(g2, src_p, dst_p)
    out = _out_call(p2, g2, degs, b2.reshape(1, D))
    return out[:N]


# speedup vs baseline: 27.2870x; 1.1584x over previous
import dataclasses
import functools

import jax
import jax.numpy as jnp
from jax import lax
from jax.experimental import pallas as pl
from jax.experimental.pallas import tpu as pltpu
from jax.experimental.pallas import tpu_sc as plsc

N = 10000
D = 128
NC = 2
NS = 16
L = 16
NW = NC * NS
K = 128
CH = 80
CP = 40
P = CH // CP
EPW = CH * K
EP = NW * EPW
NPAD = 10240
RPW = NPAD // NS
BM = 2048

_mesh = plsc.VectorSubcoreMesh(
    core_axis_name="c", subcore_axis_name="s", num_cores=NC, num_subcores=NS
)

_sc_params = pltpu.CompilerParams()
if "needs_layout_passes" in pltpu.CompilerParams.__dataclass_fields__:
    _sc_params = dataclasses.replace(_sc_params, needs_layout_passes=False)


def _deg_body(dst_hbm, deg_hbm, d_v, hist):
    c = lax.axis_index("c")
    s = lax.axis_index("s")
    w = c * NS + s
    zero16 = jnp.zeros((L,), jnp.float32)
    one16 = jnp.ones((L,), jnp.float32)

    @pl.loop(0, NPAD, step=L)
    def _(i):
        hist[pl.ds(i, L)] = zero16

    pltpu.sync_copy(dst_hbm.at[w], d_v)

    @pl.loop(0, CH)
    def _(j):
        @pl.loop(0, K, step=L)
        def _(k):
            idx = d_v[j, pl.ds(k, L)]
            plsc.addupdate_scatter(hist, [idx], one16)

    pltpu.sync_copy(hist, deg_hbm.at[w])


_deg_call = pl.kernel(
    _deg_body,
    out_type=jax.ShapeDtypeStruct((NW, NPAD), jnp.float32),
    mesh=_mesh,
    scratch_types=[
        pltpu.VMEM((CH, K), jnp.int32),
        pltpu.VMEM((NPAD,), jnp.float32),
    ],
    compiler_params=_sc_params,
)


def _edge_body(g_hbm, src_hbm, dst_hbm, part_hbm, s_v, d_v, buf_a, buf_b,
               accum, gsem_a, gsem_b, ssem_a, ssem_b):
    c = lax.axis_index("c")
    s = lax.axis_index("s")
    w = c * NS + s
    zero16 = jnp.zeros((L,), jnp.float32)

    @pl.loop(0, K)
    def _(r):
        @pl.loop(0, D, step=L)
        def _(k):
            buf_a[r, pl.ds(k, L)] = zero16

    @pl.loop(0, RPW, step=K)
    def _(r0):
        pltpu.sync_copy(buf_a, accum.at[pl.ds(s * RPW + r0, K)])

    plsc.subcore_barrier()

    @pl.loop(0, P)
    def _(p):
        pltpu.sync_copy(src_hbm.at[w, pl.ds(p * CP, CP)], s_v)
        pltpu.sync_copy(dst_hbm.at[w, pl.ds(p * CP, CP)], d_v)

        pltpu.async_copy(g_hbm.at[s_v.at[0]], buf_a, gsem_a)
        pltpu.async_copy(g_hbm.at[s_v.at[1]], buf_b, gsem_b)

        @pl.loop(0, CP, step=2)
        def _(j):
            pltpu.make_async_copy(g_hbm.at[s_v.at[j]], buf_a, gsem_a).wait()
            sc_a = pltpu.async_copy(buf_a, accum.at[d_v.at[j]], ssem_a,
                                    add=True)
            pltpu.make_async_copy(g_hbm.at[s_v.at[j + 1]], buf_b,
                                  gsem_b).wait()
            sc_b = pltpu.async_copy(buf_b, accum.at[d_v.at[j + 1]], ssem_b,
                                    add=True)
            sc_a.wait()

            @pl.when(j + 2 < CP)
            def _():
                pltpu.async_copy(g_hbm.at[s_v.at[j + 2]], buf_a, gsem_a)

            sc_b.wait()

            @pl.when(j + 3 < CP)
            def _():
                pltpu.async_copy(g_hbm.at[s_v.at[j + 3]], buf_b, gsem_b)

    plsc.subcore_barrier()

    pltpu.sync_copy(
        accum.at[pl.ds(s * RPW, RPW)],
        part_hbm.at[c, pl.ds(s * RPW, RPW)],
    )


_edge_call = pl.kernel(
    _edge_body,
    out_type=jax.ShapeDtypeStruct((NC, NPAD, D), jnp.float32),
    mesh=_mesh,
    scratch_types=[
        pltpu.VMEM((CP, K), jnp.int32),
        pltpu.VMEM((CP, K), jnp.int32),
        pltpu.VMEM((K, D), jnp.float32),
        pltpu.VMEM((K, D), jnp.float32),
        pltpu.VMEM_SHARED((NPAD, D), jnp.float32),
        pltpu.SemaphoreType.DMA,
        pltpu.SemaphoreType.DMA,
        pltpu.SemaphoreType.DMA,
        pltpu.SemaphoreType.DMA,
    ],
    compiler_params=_sc_params,
)


def _dis_of(degs_blk):
    deg = jnp.sum(degs_blk, axis=0) + 1.0
    return lax.rsqrt(deg)


def _mm1_body(x_ref, w_ref, degs_ref, g_ref):
    dis = _dis_of(degs_ref[...])
    h = jnp.dot(x_ref[...], w_ref[...], preferred_element_type=jnp.float32)
    g_ref[...] = h * dis[:, None]


_mm1_call = pl.pallas_call(
    _mm1_body,
    grid=(NPAD // BM,),
    in_specs=[
        pl.BlockSpec((BM, D), lambda i: (i, 0)),
        pl.BlockSpec((D, D), lambda i: (0, 0)),
        pl.BlockSpec((NW, BM), lambda i: (0, i)),
    ],
    out_specs=pl.BlockSpec((BM, D), lambda i: (i, 0)),
    out_shape=jax.ShapeDtypeStruct((NPAD, D), jnp.float32),
)


def _mm2_body(p_ref, g1_ref, degs_ref, b1_ref, w2_ref, g2_ref):
    dis = _dis_of(degs_ref[...])
    m = p_ref[0] + p_ref[1] + g1_ref[...]
    a = jnp.maximum(m * dis[:, None] + b1_ref[...], 0.0)
    h2 = jnp.dot(a, w2_ref[...], preferred_element_type=jnp.float32)
    g2_ref[...] = h2 * dis[:, None]


_mm2_call = pl.pallas_call(
    _mm2_body,
    grid=(NPAD // BM,),
    in_specs=[
        pl.BlockSpec((NC, BM, D), lambda i: (0, i, 0)),
        pl.BlockSpec((BM, D), lambda i: (i, 0)),
        pl.BlockSpec((NW, BM), lambda i: (0, i)),
        pl.BlockSpec((1, D), lambda i: (0, 0)),
        pl.BlockSpec((D, D), lambda i: (0, 0)),
    ],
    out_specs=pl.BlockSpec((BM, D), lambda i: (i, 0)),
    out_shape=jax.ShapeDtypeStruct((NPAD, D), jnp.float32),
)


def _out_body(p_ref, g2_ref, degs_ref, b2_ref, o_ref):
    dis = _dis_of(degs_ref[...])
    m = p_ref[0] + p_ref[1] + g2_ref[...]
    o_ref[...] = m * dis[:, None] + b2_ref[...]


_out_call = pl.pallas_call(
    _out_body,
    grid=(NPAD // BM,),
    in_specs=[
        pl.BlockSpec((NC, BM, D), lambda i: (0, i, 0)),
        pl.BlockSpec((BM, D), lambda i: (i, 0)),
        pl.BlockSpec((NW, BM), lambda i: (0, i)),
        pl.BlockSpec((1, D), lambda i: (0, 0)),
    ],
    out_specs=pl.BlockSpec((BM, D), lambda i: (i, 0)),
    out_shape=jax.ShapeDtypeStruct((NPAD, D), jnp.float32),
)


def kernel(x, edge_index, W1, b1, W2, b2):
    e = edge_index.shape[1]
    pad = EP - e
    ar = jnp.arange(pad, dtype=jnp.int32)
    src_p = jnp.concatenate([edge_index[0], ar % N]).reshape(NW, CH, K)
    dst_p = jnp.concatenate([edge_index[1], N + ar % (NPAD - N)]).reshape(
        NW, CH, K)
    x_p = jnp.pad(x, ((0, NPAD - N), (0, 0)))

    degs = _deg_call(dst_p)
    g1 = _mm1_call(x_p, W1, degs)
    p1 = _edge_call(g1, src_p, dst_p)
    g2 = _mm2_call(p1, g1, degs, b1.reshape(1, D), W2)
    p2 = _edge_call(g2, src_p, dst_p)
    out = _out_call(p2, g2, degs, b2.reshape(1, D))
    return out[:N]


# speedup vs baseline: 31.2400x; 1.1449x over previous
import dataclasses

import jax
import jax.numpy as jnp
from jax import lax
from jax.experimental import pallas as pl
from jax.experimental.pallas import tpu as pltpu
from jax.experimental.pallas import tpu_sc as plsc

N = 10000
D = 128
NC = 2
NS = 16
L = 16
NW = NC * NS
K = 64
CH = 160
CP = 40
P = CH // CP
NB = 4
EPW = CH * K
EP = NW * EPW
NPAD = 10240
RPW = NPAD // NS
BM = 2048

_mesh = plsc.VectorSubcoreMesh(
    core_axis_name="c", subcore_axis_name="s", num_cores=NC, num_subcores=NS
)

_sc_params = pltpu.CompilerParams()
if "needs_layout_passes" in pltpu.CompilerParams.__dataclass_fields__:
    _sc_params = dataclasses.replace(_sc_params, needs_layout_passes=False)


def _deg_body(dst_hbm, deg_hbm, d_v, hist):
    c = lax.axis_index("c")
    s = lax.axis_index("s")
    w = c * NS + s
    zero16 = jnp.zeros((L,), jnp.float32)
    one16 = jnp.ones((L,), jnp.float32)

    @pl.loop(0, NPAD, step=L)
    def _(i):
        hist[pl.ds(i, L)] = zero16

    pltpu.sync_copy(dst_hbm.at[w], d_v)

    @pl.loop(0, CH)
    def _(j):
        for k in range(0, K, L):
            idx = d_v[j, pl.ds(k, L)]
            plsc.addupdate_scatter(hist, [idx], one16)

    pltpu.sync_copy(hist, deg_hbm.at[w])


_deg_call = pl.kernel(
    _deg_body,
    out_type=jax.ShapeDtypeStruct((NW, NPAD), jnp.float32),
    mesh=_mesh,
    scratch_types=[
        pltpu.VMEM((CH, K), jnp.int32),
        pltpu.VMEM((NPAD,), jnp.float32),
    ],
    compiler_params=_sc_params,
)


def _edge_body(z_hbm, g_hbm, src_hbm, dst_hbm, part_hbm, s_v, d_v, bufs,
               accum, gsems, ssems):
    c = lax.axis_index("c")
    s = lax.axis_index("s")
    w = c * NS + s

    pltpu.sync_copy(z_hbm.at[pl.ds(s * RPW, RPW)],
                    accum.at[pl.ds(s * RPW, RPW)])

    plsc.subcore_barrier()

    @pl.loop(0, P)
    def _(p):
        pltpu.sync_copy(src_hbm.at[w, pl.ds(p * CP, CP)], s_v)
        pltpu.sync_copy(dst_hbm.at[w, pl.ds(p * CP, CP)], d_v)

        def _gth(jj, b):
            return pltpu.make_async_copy(
                g_hbm.at[s_v.at[jj]], bufs[b], gsems[b])

        for b in range(NB):
            _gth(b, b).start()

        @pl.loop(0, CP, step=NB)
        def _(j):
            scs = []
            for b in range(NB):
                _gth(j + b, b).wait()
                scs.append(pltpu.async_copy(
                    bufs[b], accum.at[d_v.at[j + b]], ssems[b], add=True))
                if b >= 2:
                    bp = b - 2
                    scs[bp].wait()

                    @pl.when(j + NB + bp < CP)
                    def _(bp=bp):
                        _gth(j + NB + bp, bp).start()
            for bp in range(NB - 2, NB):
                scs[bp].wait()

                @pl.when(j + NB + bp < CP)
                def _(bp=bp):
                    _gth(j + NB + bp, bp).start()

    plsc.subcore_barrier()

    pltpu.sync_copy(
        accum.at[pl.ds(s * RPW, RPW)],
        part_hbm.at[c, pl.ds(s * RPW, RPW)],
    )


_edge_call = pl.kernel(
    _edge_body,
    out_type=jax.ShapeDtypeStruct((NC, NPAD, D), jnp.float32),
    mesh=_mesh,
    scratch_types=[
        pltpu.VMEM((CP, K), jnp.int32),
        pltpu.VMEM((CP, K), jnp.int32),
        [pltpu.VMEM((K, D), jnp.float32) for _ in range(NB)],
        pltpu.VMEM_SHARED((NPAD, D), jnp.float32),
        [pltpu.SemaphoreType.DMA for _ in range(NB)],
        [pltpu.SemaphoreType.DMA for _ in range(NB)],
    ],
    compiler_params=_sc_params,
)


def _dis_of(degs_blk):
    deg = jnp.sum(degs_blk, axis=0) + 1.0
    return lax.rsqrt(deg)


def _mm1_body(x_ref, w_ref, degs_ref, g_ref):
    dis = _dis_of(degs_ref[...])
    h = jnp.dot(x_ref[...], w_ref[...], preferred_element_type=jnp.float32)
    g_ref[...] = h * dis[:, None]


_mm1_call = pl.pallas_call(
    _mm1_body,
    grid=(NPAD // BM,),
    in_specs=[
        pl.BlockSpec((BM, D), lambda i: (i, 0)),
        pl.BlockSpec((D, D), lambda i: (0, 0)),
        pl.BlockSpec((NW, BM), lambda i: (0, i)),
    ],
    out_specs=pl.BlockSpec((BM, D), lambda i: (i, 0)),
    out_shape=jax.ShapeDtypeStruct((NPAD, D), jnp.float32),
)


def _mm2_body(p_ref, g1_ref, degs_ref, b1_ref, w2_ref, g2_ref):
    dis = _dis_of(degs_ref[...])
    m = p_ref[0] + p_ref[1] + g1_ref[...]
    a = jnp.maximum(m * dis[:, None] + b1_ref[...], 0.0)
    h2 = jnp.dot(a, w2_ref[...], preferred_element_type=jnp.float32)
    g2_ref[...] = h2 * dis[:, None]


_mm2_call = pl.pallas_call(
    _mm2_body,
    grid=(NPAD // BM,),
    in_specs=[
        pl.BlockSpec((NC, BM, D), lambda i: (0, i, 0)),
        pl.BlockSpec((BM, D), lambda i: (i, 0)),
        pl.BlockSpec((NW, BM), lambda i: (0, i)),
        pl.BlockSpec((1, D), lambda i: (0, 0)),
        pl.BlockSpec((D, D), lambda i: (0, 0)),
    ],
    out_specs=pl.BlockSpec((BM, D), lambda i: (i, 0)),
    out_shape=jax.ShapeDtypeStruct((NPAD, D), jnp.float32),
)


def _out_body(p_ref, g2_ref, degs_ref, b2_ref, o_ref):
    dis = _dis_of(degs_ref[...])
    m = p_ref[0] + p_ref[1] + g2_ref[...]
    o_ref[...] = m * dis[:, None] + b2_ref[...]


_out_call = pl.pallas_call(
    _out_body,
    grid=(NPAD // BM,),
    in_specs=[
        pl.BlockSpec((NC, BM, D), lambda i: (0, i, 0)),
        pl.BlockSpec((BM, D), lambda i: (i, 0)),
        pl.BlockSpec((NW, BM), lambda i: (0, i)),
        pl.BlockSpec((1, D), lambda i: (0, 0)),
    ],
    out_specs=pl.BlockSpec((BM, D), lambda i: (i, 0)),
    out_shape=jax.ShapeDtypeStruct((NPAD, D), jnp.float32),
)


def kernel(x, edge_index, W1, b1, W2, b2):
    e = edge_index.shape[1]
    pad = EP - e
    ar = jnp.arange(pad, dtype=jnp.int32)
    src_p = jnp.concatenate([edge_index[0], ar % N]).reshape(NW, CP * P, K)
    dst_p = jnp.concatenate([edge_index[1], N + ar % (NPAD - N)]).reshape(
        NW, CP * P, K)
    x_p = jnp.pad(x, ((0, NPAD - N), (0, 0)))

    zeros = jnp.zeros((NPAD, D), jnp.float32)

    degs = _deg_call(dst_p)
    g1 = _mm1_call(x_p, W1, degs)
    p1 = _edge_call(zeros, g1, src_p, dst_p)
    g2 = _mm2_call(p1, g1, degs, b1.reshape(1, D), W2)
    p2 = _edge_call(zeros, g2, src_p, dst_p)
    out = _out_call(p2, g2, degs, b2.reshape(1, D))
    return out[:N]


# speedup vs baseline: 31.9887x; 1.0240x over previous
import dataclasses

import jax
import jax.numpy as jnp
from jax import lax
from jax.experimental import pallas as pl
from jax.experimental.pallas import tpu as pltpu
from jax.experimental.pallas import tpu_sc as plsc

N = 10000
D = 128
NC = 2
NS = 16
L = 16
NW = NC * NS
K = 64
CH = 160
CP = 40
P = CH // CP
NB = 4
EPW = CH * K
EP = NW * EPW
NPAD = 10240
RPW = NPAD // NS
BM = 2048

_mesh = plsc.VectorSubcoreMesh(
    core_axis_name="c", subcore_axis_name="s", num_cores=NC, num_subcores=NS
)

_sc_params = pltpu.CompilerParams()
if "needs_layout_passes" in pltpu.CompilerParams.__dataclass_fields__:
    _sc_params = dataclasses.replace(_sc_params, needs_layout_passes=False)


def _deg_body(dst_hbm, deg_hbm, d_v, hist):
    c = lax.axis_index("c")
    s = lax.axis_index("s")
    w = c * NS + s
    zero16 = jnp.zeros((L,), jnp.float32)
    one16 = jnp.ones((L,), jnp.float32)

    @pl.loop(0, NPAD, step=L)
    def _(i):
        hist[pl.ds(i, L)] = zero16

    pltpu.sync_copy(dst_hbm.at[w], d_v)

    @pl.loop(0, CH)
    def _(j):
        for k in range(0, K, L):
            idx = d_v[j, pl.ds(k, L)]
            plsc.addupdate_scatter(hist, [idx], one16)

    pltpu.sync_copy(hist, deg_hbm.at[w])


_deg_call = pl.kernel(
    _deg_body,
    out_type=jax.ShapeDtypeStruct((NW, NPAD), jnp.float32),
    mesh=_mesh,
    scratch_types=[
        pltpu.VMEM((CH, K), jnp.int32),
        pltpu.VMEM((NPAD,), jnp.float32),
    ],
    compiler_params=_sc_params,
)


def _edge_body(g_hbm, src_hbm, dst_hbm, part_hbm, s_v, d_v, bufs, accum,
               gsems, ssems):
    c = lax.axis_index("c")
    s = lax.axis_index("s")
    w = c * NS + s
    zero16 = jnp.zeros((L,), jnp.float32)

    @pl.loop(0, K)
    def _(r):
        for k in range(0, D, L):
            bufs[0][r, pl.ds(k, L)] = zero16

    @pl.loop(0, RPW, step=K)
    def _(r0):
        pltpu.sync_copy(bufs[0], accum.at[pl.ds(s * RPW + r0, K)])

    plsc.subcore_barrier()

    @pl.loop(0, P)
    def _(p):
        pltpu.sync_copy(src_hbm.at[w, pl.ds(p * CP, CP)], s_v)
        pltpu.sync_copy(dst_hbm.at[w, pl.ds(p * CP, CP)], d_v)

        def _gth(jj, b):
            return pltpu.make_async_copy(
                g_hbm.at[s_v.at[jj]], bufs[b], gsems[b])

        for b in range(NB):
            _gth(b, b).start()

        @pl.loop(0, CP, step=NB)
        def _(j):
            scs = []
            for b in range(NB):
                _gth(j + b, b).wait()
                scs.append(pltpu.async_copy(
                    bufs[b], accum.at[d_v.at[j + b]], ssems[b], add=True))
                if b >= 2:
                    bp = b - 2
                    scs[bp].wait()

                    @pl.when(j + NB + bp < CP)
                    def _(bp=bp):
                        _gth(j + NB + bp, bp).start()
            for bp in range(NB - 2, NB):
                scs[bp].wait()

                @pl.when(j + NB + bp < CP)
                def _(bp=bp):
                    _gth(j + NB + bp, bp).start()

    plsc.subcore_barrier()

    pltpu.sync_copy(
        accum.at[pl.ds(s * RPW, RPW)],
        part_hbm.at[c, pl.ds(s * RPW, RPW)],
    )


_edge_call = pl.kernel(
    _edge_body,
    out_type=jax.ShapeDtypeStruct((NC, NPAD, D), jnp.float32),
    mesh=_mesh,
    scratch_types=[
        pltpu.VMEM((CP, K), jnp.int32),
        pltpu.VMEM((CP, K), jnp.int32),
        [pltpu.VMEM((K, D), jnp.float32) for _ in range(NB)],
        pltpu.VMEM_SHARED((NPAD, D), jnp.float32),
        [pltpu.SemaphoreType.DMA for _ in range(NB)],
        [pltpu.SemaphoreType.DMA for _ in range(NB)],
    ],
    compiler_params=_sc_params,
)


def _dis_of(degs_blk):
    deg = jnp.sum(degs_blk, axis=0) + 1.0
    return lax.rsqrt(deg)


def _mm1_body(x_ref, w_ref, degs_ref, g_ref):
    dis = _dis_of(degs_ref[...])
    h = jnp.dot(x_ref[...], w_ref[...], preferred_element_type=jnp.float32)
    g_ref[...] = h * dis[:, None]


_mm1_call = pl.pallas_call(
    _mm1_body,
    grid=(NPAD // BM,),
    in_specs=[
        pl.BlockSpec((BM, D), lambda i: (i, 0)),
        pl.BlockSpec((D, D), lambda i: (0, 0)),
        pl.BlockSpec((NW, BM), lambda i: (0, i)),
    ],
    out_specs=pl.BlockSpec((BM, D), lambda i: (i, 0)),
    out_shape=jax.ShapeDtypeStruct((NPAD, D), jnp.float32),
)


def _mm2_body(p_ref, g1_ref, degs_ref, b1_ref, w2_ref, g2_ref):
    dis = _dis_of(degs_ref[...])
    m = p_ref[0] + p_ref[1] + g1_ref[...]
    a = jnp.maximum(m * dis[:, None] + b1_ref[...], 0.0)
    h2 = jnp.dot(a, w2_ref[...], preferred_element_type=jnp.float32)
    g2_ref[...] = h2 * dis[:, None]


_mm2_call = pl.pallas_call(
    _mm2_body,
    grid=(NPAD // BM,),
    in_specs=[
        pl.BlockSpec((NC, BM, D), lambda i: (0, i, 0)),
        pl.BlockSpec((BM, D), lambda i: (i, 0)),
        pl.BlockSpec((NW, BM), lambda i: (0, i)),
        pl.BlockSpec((1, D), lambda i: (0, 0)),
        pl.BlockSpec((D, D), lambda i: (0, 0)),
    ],
    out_specs=pl.BlockSpec((BM, D), lambda i: (i, 0)),
    out_shape=jax.ShapeDtypeStruct((NPAD, D), jnp.float32),
)


def _out_body(p_ref, g2_ref, degs_ref, b2_ref, o_ref):
    dis = _dis_of(degs_ref[...])
    m = p_ref[0] + p_ref[1] + g2_ref[...]
    o_ref[...] = m * dis[:, None] + b2_ref[...]


_out_call = pl.pallas_call(
    _out_body,
    grid=(NPAD // BM,),
    in_specs=[
        pl.BlockSpec((NC, BM, D), lambda i: (0, i, 0)),
        pl.BlockSpec((BM, D), lambda i: (i, 0)),
        pl.BlockSpec((NW, BM), lambda i: (0, i)),
        pl.BlockSpec((1, D), lambda i: (0, 0)),
    ],
    out_specs=pl.BlockSpec((BM, D), lambda i: (i, 0)),
    out_shape=jax.ShapeDtypeStruct((NPAD, D), jnp.float32),
)


def kernel(x, edge_index, W1, b1, W2, b2):
    e = edge_index.shape[1]
    pad = EP - e
    ar = jnp.arange(pad, dtype=jnp.int32)
    src_p = jnp.concatenate([edge_index[0], ar % N]).reshape(NW, CP * P, K)
    dst_p = jnp.concatenate([edge_index[1], N + ar % (NPAD - N)]).reshape(
        NW, CP * P, K)
    x_p = jnp.pad(x, ((0, NPAD - N), (0, 0)))

    degs = _deg_call(dst_p)
    g1 = _mm1_call(x_p, W1, degs)
    p1 = _edge_call(g1, src_p, dst_p)
    g2 = _mm2_call(p1, g1, degs, b1.reshape(1, D), W2)
    p2 = _edge_call(g2, src_p, dst_p)
    out = _out_call(p2, g2, degs, b2.reshape(1, D))
    return out[:N]
